# baseline (speedup 1.0000x reference)
import functools

import jax
import jax.numpy as jnp
from jax import lax
from jax.experimental import pallas as pl
from jax.experimental.pallas import tpu as pltpu
from jax.experimental.pallas import tpu_sc as plsc

HID = 768
ENT_D = 256
LN_EPS = 1e-12

_NC = 2
_NS = 16
_NW = _NC * _NS

_RB = 2048


def _make_sc_gather(n: int, d: int, chunk: int, dtype=jnp.float32):
    per_w = n // _NW
    n_chunks = per_w // chunk
    assert per_w % chunk == 0 and chunk % 8 == 0 and chunk <= 128

    mesh = plsc.VectorSubcoreMesh(core_axis_name="c", subcore_axis_name="s",
                                  num_cores=_NC, num_subcores=_NS)

    @functools.partial(
        pl.kernel,
        out_type=jax.ShapeDtypeStruct((n, d), dtype),
        mesh=mesh,
        scratch_types=[
            pltpu.VMEM((per_w,), jnp.int32),
            pltpu.VMEM((2, chunk, d), dtype),
            pltpu.SemaphoreType.DMA,
            pltpu.SemaphoreType.DMA,
        ],
    )
    def k(table_hbm, idx_hbm, out_hbm, idx_v, buf, sem0, sem1):
        wid = lax.axis_index("s") * _NC + lax.axis_index("c")
        base = wid * per_w
        pltpu.sync_copy(idx_hbm.at[pl.ds(base, per_w)], idx_v)

        def gather(g, slot, sem):
            return pltpu.async_copy(
                table_hbm.at[idx_v.at[pl.ds(g * chunk, chunk)]],
                buf.at[slot], sem)

        def gwait(slot, sem):
            pltpu.make_async_copy(
                table_hbm.at[idx_v.at[pl.ds(0, chunk)]], buf.at[slot], sem
            ).wait()

        gather(0, 0, sem0)

        def body(i, carry):
            g0 = 2 * i

            @pl.when(g0 + 1 < n_chunks)
            def _():
                gather(g0 + 1, 1, sem1)

            gwait(0, sem0)
            pltpu.sync_copy(buf.at[0],
                            out_hbm.at[pl.ds(base + g0 * chunk, chunk)])

            @pl.when(g0 + 2 < n_chunks)
            def _():
                gather(g0 + 2, 0, sem0)

            @pl.when(g0 + 1 < n_chunks)
            def _():
                gwait(1, sem1)
                pltpu.sync_copy(
                    buf.at[1],
                    out_hbm.at[pl.ds(base + (g0 + 1) * chunk, chunk)])

            return carry

        lax.fori_loop(0, (n_chunks + 1) // 2, body, 0)

    return k


def _tc_body(word_ref, ent_ref, stat_ref, tt_ref, pos_ref, tokd_ref,
             pe_ref, ps_ref, g_ref, b_ref, out_ref):
    dn = (((1,), (1,)), ((), ()))
    e = lax.dot_general(ent_ref[...].astype(jnp.bfloat16), pe_ref[...], dn,
                        preferred_element_type=jnp.float32)
    s = lax.dot_general(stat_ref[...].astype(jnp.bfloat16), ps_ref[...], dn,
                        preferred_element_type=jnp.float32)
    wu = lax.bitcast_convert_type(word_ref[...], jnp.uint32)
    hi_f = lax.bitcast_convert_type(wu & jnp.uint32(0xFFFF0000), jnp.float32)
    lo_f = lax.bitcast_convert_type(wu << 16, jnp.float32)
    word_f = jnp.concatenate([hi_f, lo_f], axis=1)
    x = word_f + tt_ref[...] * tokd_ref[...] + e + s
    rows, cols = x.shape
    x = (x.reshape(rows // 512, 512, cols) + pos_ref[...][None]
         ).reshape(rows, cols)
    mean = jnp.mean(x, axis=1, keepdims=True)
    xc = x - mean
    var = jnp.mean(xc * xc, axis=1, keepdims=True)
    out_ref[...] = xc * lax.rsqrt(var + LN_EPS) * g_ref[...] + b_ref[...]


def _tc_fuse(word_rows, ent_rows, stat_rows, tt_col, pos_plus, tok_delta,
             proj_e, proj_s, gamma_row, beta_row):
    n = word_rows.shape[0]
    rb = _RB
    return pl.pallas_call(
        _tc_body,
        grid=(n // rb,),
        in_specs=[
            pl.BlockSpec((rb, HID // 2), lambda i: (i, 0)),
            pl.BlockSpec((rb, ENT_D), lambda i: (i, 0)),
            pl.BlockSpec((rb, ENT_D), lambda i: (i, 0)),
            pl.BlockSpec((rb, 1), lambda i: (i, 0)),
            pl.BlockSpec((512, HID), lambda i: (0, 0)),
            pl.BlockSpec((1, HID), lambda i: (0, 0)),
            pl.BlockSpec((HID, ENT_D), lambda i: (0, 0)),
            pl.BlockSpec((HID, ENT_D), lambda i: (0, 0)),
            pl.BlockSpec((1, HID), lambda i: (0, 0)),
            pl.BlockSpec((1, HID), lambda i: (0, 0)),
        ],
        out_specs=pl.BlockSpec((rb, HID), lambda i: (i, 0)),
        out_shape=jax.ShapeDtypeStruct((n, HID), jnp.float32),
    )(word_rows, ent_rows, stat_rows, tt_col, pos_plus, tok_delta,
      proj_e, proj_s, gamma_row, beta_row)


def kernel(input_ids, input_ent_ids, input_static_ent_ids, token_type_ids,
           word_emb, pos_emb, tok_emb, ent_emb, ent_proj,
           static_ent_emb, static_ent_proj, ln_gamma, ln_beta):
    b, s = input_ids.shape
    n = b * s

    ids = input_ids.reshape(n).astype(jnp.int32)
    eids = input_ent_ids.reshape(n).astype(jnp.int32)
    sids = input_static_ent_ids.reshape(n).astype(jnp.int32)
    tt_col = token_type_ids.reshape(n, 1).astype(jnp.float32)

    pos_plus = pos_emb + tok_emb[0][None, :]
    tok_delta = (tok_emb[1] - tok_emb[0])[None, :]
    proj_e_bf = ent_proj.astype(jnp.bfloat16)
    proj_s_bf = static_ent_proj.astype(jnp.bfloat16)

    wb = word_emb.astype(jnp.bfloat16)
    hi = lax.bitcast_convert_type(wb[:, :HID // 2], jnp.uint16).astype(jnp.uint32)
    lo = lax.bitcast_convert_type(wb[:, HID // 2:], jnp.uint16).astype(jnp.uint32)
    word_i32 = lax.bitcast_convert_type((hi << 16) | lo, jnp.int32)

    w_rows = _make_sc_gather(n, HID // 2, 128, jnp.int32)(word_i32, ids)
    e_rows = _make_sc_gather(n, ENT_D, 128)(ent_emb, eids)
    s_rows = _make_sc_gather(n, ENT_D, 128)(static_ent_emb, sids)

    out = _tc_fuse(w_rows, e_rows, s_rows, tt_col, pos_plus, tok_delta,
                   proj_e_bf, proj_s_bf, ln_gamma[None, :], ln_beta[None, :])
    return out.reshape(b, s, HID)
